# baseline (speedup 1.0000x reference)
import functools

import jax
import jax.numpy as jnp
from jax import lax
from jax.experimental import pallas as pl
from jax.experimental.pallas import tpu as pltpu
from jax.experimental.pallas import tpu_sc as plsc

D_MODEL = 64
NUM_WORKERS = 32
CHUNK = 512


def _make_gather(b_total):
    b_per_w = b_total // NUM_WORKERS
    n_chunks = b_per_w // CHUNK
    mesh = plsc.VectorSubcoreMesh(core_axis_name="c", subcore_axis_name="s")

    @functools.partial(
        pl.kernel,
        mesh=mesh,
        out_type=jax.ShapeDtypeStruct((b_total, D_MODEL), jnp.float32),
        scratch_types=[
            pltpu.VMEM((CHUNK,), jnp.int32),
            pltpu.VMEM((CHUNK, D_MODEL), jnp.float32),
            pltpu.SemaphoreType.DMA,
        ],
        compiler_params=pltpu.CompilerParams(use_tc_tiling_on_sc=False),
    )
    def gather_kernel(idx_hbm, table_hbm, out_hbm, idx_v, rows_v, sem):
        wid = lax.axis_index("s") * 2 + lax.axis_index("c")
        base = wid * b_per_w

        def body(g, carry):
            off = base + g * CHUNK
            pltpu.sync_copy(idx_hbm.at[pl.ds(off, CHUNK)], idx_v)
            pltpu.async_copy(table_hbm.at[idx_v], rows_v, sem).wait()
            pltpu.sync_copy(rows_v, out_hbm.at[pl.ds(off, CHUNK)])
            return carry

        lax.fori_loop(0, n_chunks, body, 0)

    return gather_kernel


def kernel(tokens, token_emb):
    batch, seq = tokens.shape
    idx = tokens.reshape(-1).astype(jnp.int32)
    out = _make_gather(batch * seq)(idx, token_emb)
    return out.reshape(batch, seq, D_MODEL)


# speedup vs baseline: 1.0434x; 1.0434x over previous
import functools

import jax
import jax.numpy as jnp
from jax import lax
from jax.experimental import pallas as pl
from jax.experimental.pallas import tpu as pltpu
from jax.experimental.pallas import tpu_sc as plsc

D_MODEL = 64
NUM_WORKERS = 32
CHUNK = 400
NBUF = 4
LAG = 2


def _make_gather(b_total):
    b_per_w = b_total // NUM_WORKERS
    n_chunks = b_per_w // CHUNK
    assert n_chunks % NBUF == 0 and n_chunks // NBUF >= 2
    mesh = plsc.VectorSubcoreMesh(core_axis_name="c", subcore_axis_name="s")

    @functools.partial(
        pl.kernel,
        mesh=mesh,
        out_type=jax.ShapeDtypeStruct((b_total, D_MODEL), jnp.float32),
        scratch_types=[
            pltpu.VMEM((b_per_w,), jnp.int32),
            [pltpu.VMEM((CHUNK, D_MODEL), jnp.float32) for _ in range(NBUF)],
            [pltpu.SemaphoreType.DMA for _ in range(NBUF)],
            [pltpu.SemaphoreType.DMA for _ in range(NBUF)],
        ],
        compiler_params=pltpu.CompilerParams(use_tc_tiling_on_sc=False),
    )
    def gather_kernel(idx_hbm, table_hbm, out_hbm, idx_v, rows, sg, so):
        wid = lax.axis_index("s") * 2 + lax.axis_index("c")
        base = wid * b_per_w
        pltpu.sync_copy(idx_hbm.at[pl.ds(base, b_per_w)], idx_v)

        def issue_gather(g, b):
            pltpu.async_copy(
                table_hbm.at[idx_v.at[pl.ds(g * CHUNK, CHUNK)]], rows[b], sg[b]
            )

        def wait_gather(b):
            pltpu.make_async_copy(
                out_hbm.at[pl.ds(base, CHUNK)], rows[b], sg[b]
            ).wait()

        def issue_store(g, b):
            pltpu.async_copy(
                rows[b], out_hbm.at[pl.ds(base + g * CHUNK, CHUNK)], so[b]
            )

        def wait_store(b):
            pltpu.make_async_copy(
                rows[b], out_hbm.at[pl.ds(base, CHUNK)], so[b]
            ).wait()

        for b in range(NBUF):
            issue_gather(b, b)
            if b >= LAG:
                wait_gather(b - LAG)
                issue_store(b - LAG, b - LAG)

        def body(outer, carry):
            for b in range(NBUF):
                g = outer * NBUF + b
                wait_store(b)
                issue_gather(g, b)
                b2 = (b - LAG) % NBUF
                wait_gather(b2)
                issue_store(g - LAG, b2)
            return carry

        lax.fori_loop(1, n_chunks // NBUF, body, 0)

        for i in range(LAG):
            b = (NBUF - LAG + i) % NBUF
            wait_gather(b)
            issue_store(n_chunks - LAG + i, b)
        for b in range(NBUF):
            wait_store(b)

    return gather_kernel


def kernel(tokens, token_emb):
    batch, seq = tokens.shape
    idx = tokens.reshape(-1).astype(jnp.int32)
    out = _make_gather(batch * seq)(idx, token_emb)
    return out.reshape(batch, seq, D_MODEL)
